# baseline (device time: 1437667 ns/iter reference)
import jax
import jax.numpy as jnp
from jax import lax
from jax.experimental import pallas as pl
from jax.experimental.pallas import tpu as pltpu

N_DEV = 16


def kernel(x, w_mat, scale_x, scale_w):
    m_total, k = x.shape
    _, n = w_mat.shape
    m_per = m_total // N_DEV

    def body(x_ref, w_ref, sx_ref, sw_ref, out_ref,
             comm_ref, send_sems, recv_sems, credit_sem):
        my = lax.axis_index("i")
        right = (my + 1) % N_DEV
        left = (my - 1) % N_DEV

        barrier_sem = pltpu.get_barrier_semaphore()
        for nbr in (left, right):
            pl.semaphore_signal(
                barrier_sem, inc=1,
                device_id=(nbr,), device_id_type=pl.DeviceIdType.MESH,
            )
        pl.semaphore_wait(barrier_sem, 2)

        def partial_for(chunk_idx):
            rows = x_ref[pl.ds(chunk_idx * m_per, m_per), :]
            return lax.dot_general(
                rows, w_ref[:, :],
                (((1,), (0,)), ((), ())),
                preferred_element_type=jnp.int32,
            )

        comm_ref[0, :, :] = partial_for((my - 1) % N_DEV)

        for h in range(N_DEV - 1):
            s_slot = h % 2
            r_slot = (h + 1) % 2
            if h >= 1:
                pl.semaphore_wait(credit_sem, 1)
            rdma = pltpu.make_async_remote_copy(
                src_ref=comm_ref.at[s_slot],
                dst_ref=comm_ref.at[r_slot],
                send_sem=send_sems.at[s_slot],
                recv_sem=recv_sems.at[r_slot],
                device_id=(right,),
                device_id_type=pl.DeviceIdType.MESH,
            )
            rdma.start()
            part = partial_for((my - 2 - h) % N_DEV)
            rdma.wait()
            if h < N_DEV - 2:
                pl.semaphore_signal(
                    credit_sem, inc=1,
                    device_id=(left,), device_id_type=pl.DeviceIdType.MESH,
                )
            comm_ref[r_slot, :, :] = comm_ref[r_slot, :, :] + part

        scale = sx_ref[0] * sw_ref[0]
        acc = comm_ref[(N_DEV - 1) % 2, :, :]
        out_ref[:, :] = jnp.maximum(acc.astype(jnp.float32) * scale, 0.0)

    return pl.pallas_call(
        body,
        out_shape=jax.ShapeDtypeStruct((m_per, n), jnp.float32),
        in_specs=[
            pl.BlockSpec(memory_space=pltpu.VMEM),
            pl.BlockSpec(memory_space=pltpu.VMEM),
            pl.BlockSpec(memory_space=pltpu.SMEM),
            pl.BlockSpec(memory_space=pltpu.SMEM),
        ],
        out_specs=pl.BlockSpec(memory_space=pltpu.VMEM),
        scratch_shapes=[
            pltpu.VMEM((2, m_per, n), jnp.int32),
            pltpu.SemaphoreType.DMA((2,)),
            pltpu.SemaphoreType.DMA((2,)),
            pltpu.SemaphoreType.REGULAR,
        ],
        compiler_params=pltpu.CompilerParams(collective_id=0),
    )(x, w_mat, scale_x, scale_w)


# device time: 763463 ns/iter; 1.8831x vs baseline; 1.8831x over previous
import jax
import jax.numpy as jnp
from jax import lax
from jax.experimental import pallas as pl
from jax.experimental.pallas import tpu as pltpu

N_DEV = 16


def kernel(x, w_mat, scale_x, scale_w):
    m_total, k = x.shape
    _, n = w_mat.shape
    m_per = m_total // N_DEV
    n2 = n // 2

    def body(x_ref, w_ref, sx_ref, sw_ref, out_ref,
             comm_r, comm_l, send_sems_r, recv_sems_r, send_sems_l,
             recv_sems_l, credit_r, credit_l):
        my = lax.axis_index("i")
        right = (my + 1) % N_DEV
        left = (my - 1) % N_DEV

        barrier_sem = pltpu.get_barrier_semaphore()
        for nbr in (left, right):
            pl.semaphore_signal(
                barrier_sem, inc=1,
                device_id=(nbr,), device_id_type=pl.DeviceIdType.MESH,
            )
        pl.semaphore_wait(barrier_sem, 2)

        def partial_r(chunk_idx):
            rows = x_ref[pl.ds(chunk_idx * m_per, m_per), :]
            return lax.dot_general(
                rows, w_ref[:, :n2],
                (((1,), (0,)), ((), ())),
                preferred_element_type=jnp.int32,
            )

        def partial_l(chunk_idx):
            rows = x_ref[pl.ds(chunk_idx * m_per, m_per), :]
            return lax.dot_general(
                rows, w_ref[:, n2:],
                (((1,), (0,)), ((), ())),
                preferred_element_type=jnp.int32,
            )

        comm_r[0, :, :] = partial_r((my - 1) % N_DEV)
        comm_l[0, :, :] = partial_l((my + 1) % N_DEV)

        for h in range(N_DEV - 1):
            s_slot = h % 2
            r_slot = (h + 1) % 2
            if h >= 1:
                pl.semaphore_wait(credit_r, 1)
                pl.semaphore_wait(credit_l, 1)
            rdma_r = pltpu.make_async_remote_copy(
                src_ref=comm_r.at[s_slot],
                dst_ref=comm_r.at[r_slot],
                send_sem=send_sems_r.at[s_slot],
                recv_sem=recv_sems_r.at[r_slot],
                device_id=(right,),
                device_id_type=pl.DeviceIdType.MESH,
            )
            rdma_l = pltpu.make_async_remote_copy(
                src_ref=comm_l.at[s_slot],
                dst_ref=comm_l.at[r_slot],
                send_sem=send_sems_l.at[s_slot],
                recv_sem=recv_sems_l.at[r_slot],
                device_id=(left,),
                device_id_type=pl.DeviceIdType.MESH,
            )
            rdma_r.start()
            rdma_l.start()
            part_r = partial_r((my - 2 - h) % N_DEV)
            part_l = partial_l((my + 2 + h) % N_DEV)
            rdma_r.wait()
            rdma_l.wait()
            if h < N_DEV - 2:
                pl.semaphore_signal(
                    credit_r, inc=1,
                    device_id=(left,), device_id_type=pl.DeviceIdType.MESH,
                )
                pl.semaphore_signal(
                    credit_l, inc=1,
                    device_id=(right,), device_id_type=pl.DeviceIdType.MESH,
                )
            comm_r[r_slot, :, :] = comm_r[r_slot, :, :] + part_r
            comm_l[r_slot, :, :] = comm_l[r_slot, :, :] + part_l

        scale = sx_ref[0] * sw_ref[0]
        f = (N_DEV - 1) % 2
        out_ref[:, :n2] = jnp.maximum(
            comm_r[f, :, :].astype(jnp.float32) * scale, 0.0)
        out_ref[:, n2:] = jnp.maximum(
            comm_l[f, :, :].astype(jnp.float32) * scale, 0.0)

    return pl.pallas_call(
        body,
        out_shape=jax.ShapeDtypeStruct((m_per, n), jnp.float32),
        in_specs=[
            pl.BlockSpec(memory_space=pltpu.VMEM),
            pl.BlockSpec(memory_space=pltpu.VMEM),
            pl.BlockSpec(memory_space=pltpu.SMEM),
            pl.BlockSpec(memory_space=pltpu.SMEM),
        ],
        out_specs=pl.BlockSpec(memory_space=pltpu.VMEM),
        scratch_shapes=[
            pltpu.VMEM((2, m_per, n2), jnp.int32),
            pltpu.VMEM((2, m_per, n2), jnp.int32),
            pltpu.SemaphoreType.DMA((2,)),
            pltpu.SemaphoreType.DMA((2,)),
            pltpu.SemaphoreType.DMA((2,)),
            pltpu.SemaphoreType.DMA((2,)),
            pltpu.SemaphoreType.REGULAR,
            pltpu.SemaphoreType.REGULAR,
        ],
        compiler_params=pltpu.CompilerParams(collective_id=0),
    )(x, w_mat, scale_x, scale_w)


# device time: 691525 ns/iter; 2.0790x vs baseline; 1.1040x over previous
import jax
import jax.numpy as jnp
from jax import lax
from jax.experimental import pallas as pl
from jax.experimental.pallas import tpu as pltpu

N_DEV = 16
S = 2


def kernel(x, w_mat, scale_x, scale_w):
    m_total, k = x.shape
    _, n = w_mat.shape
    m_per = m_total // N_DEV
    n2 = n // 2
    n4 = n2 // S

    def body(x_ref, w_ref, sx_ref, sw_ref, out_ref,
             comm_r, comm_l, ssem_r, rsem_r, ssem_l, rsem_l,
             credit_r, credit_l):
        my = lax.axis_index("i")
        right = (my + 1) % N_DEV
        left = (my - 1) % N_DEV

        comms = (comm_r, comm_l)
        ssems = (ssem_r, ssem_l)
        rsems = (rsem_r, rsem_l)
        credits = (credit_r, credit_l)
        dsts = (right, left)
        srcs = (left, right)

        barrier_sem = pltpu.get_barrier_semaphore()
        for nbr in (left, right):
            pl.semaphore_signal(
                barrier_sem, inc=1,
                device_id=(nbr,), device_id_type=pl.DeviceIdType.MESH,
            )
        pl.semaphore_wait(barrier_sem, 2)

        def partial(chunk_idx, ring, j):
            rows = x_ref[pl.ds(chunk_idx * m_per, m_per), :]
            col0 = ring * n2 + j * n4
            return lax.dot_general(
                rows, w_ref[:, col0:col0 + n4],
                (((1,), (0,)), ((), ())),
                preferred_element_type=jnp.int32,
            )

        def chunk_at(ring, h):
            if ring == 0:
                return (my - 2 - h) % N_DEV
            return (my + 2 + h) % N_DEV

        def desc(ring, s_slot, r_slot, j):
            return pltpu.make_async_remote_copy(
                src_ref=comms[ring].at[s_slot, j],
                dst_ref=comms[ring].at[r_slot, j],
                send_sem=ssems[ring].at[s_slot, j],
                recv_sem=rsems[ring].at[r_slot, j],
                device_id=(dsts[ring],),
                device_id_type=pl.DeviceIdType.MESH,
            )

        for ring in (0, 1):
            for j in range(S):
                comms[ring][0, j, :, :] = partial(chunk_at(ring, -1), ring, j)
        for j in range(S):
            for ring in (0, 1):
                desc(ring, 0, 1, j).start()

        for h in range(N_DEV - 1):
            s_slot = h % 2
            r_slot = (h + 1) % 2
            for j in range(S):
                for ring in (0, 1):
                    part = partial(chunk_at(ring, h), ring, j)
                    desc(ring, s_slot, r_slot, j).wait()
                    if h < N_DEV - 2:
                        pl.semaphore_signal(
                            credits[ring], inc=1,
                            device_id=(srcs[ring],),
                            device_id_type=pl.DeviceIdType.MESH,
                        )
                    comms[ring][r_slot, j, :, :] = (
                        comms[ring][r_slot, j, :, :] + part)
                    if h < N_DEV - 2:
                        pl.semaphore_wait(credits[ring], 1)
                        desc(ring, r_slot, s_slot, j).start()

        scale = sx_ref[0] * sw_ref[0]
        f = (N_DEV - 1) % 2
        for ring in (0, 1):
            for j in range(S):
                col0 = ring * n2 + j * n4
                out_ref[:, col0:col0 + n4] = jnp.maximum(
                    comms[ring][f, j, :, :].astype(jnp.float32) * scale, 0.0)

    return pl.pallas_call(
        body,
        out_shape=jax.ShapeDtypeStruct((m_per, n), jnp.float32),
        in_specs=[
            pl.BlockSpec(memory_space=pltpu.VMEM),
            pl.BlockSpec(memory_space=pltpu.VMEM),
            pl.BlockSpec(memory_space=pltpu.SMEM),
            pl.BlockSpec(memory_space=pltpu.SMEM),
        ],
        out_specs=pl.BlockSpec(memory_space=pltpu.VMEM),
        scratch_shapes=[
            pltpu.VMEM((2, S, m_per, n4), jnp.int32),
            pltpu.VMEM((2, S, m_per, n4), jnp.int32),
            pltpu.SemaphoreType.DMA((2, S)),
            pltpu.SemaphoreType.DMA((2, S)),
            pltpu.SemaphoreType.DMA((2, S)),
            pltpu.SemaphoreType.DMA((2, S)),
            pltpu.SemaphoreType.REGULAR,
            pltpu.SemaphoreType.REGULAR,
        ],
        compiler_params=pltpu.CompilerParams(collective_id=0),
    )(x, w_mat, scale_x, scale_w)


# device time: 690221 ns/iter; 2.0829x vs baseline; 1.0019x over previous
import jax
import jax.numpy as jnp
from jax import lax
from jax.experimental import pallas as pl
from jax.experimental.pallas import tpu as pltpu

N_DEV = 16
S = 2


def kernel(x, w_mat, scale_x, scale_w):
    m_total, k = x.shape
    _, n = w_mat.shape
    m_per = m_total // N_DEV
    n2 = n // 2
    n4 = n2 // S

    def body(x_ref, w_ref, sx_ref, sw_ref, out_ref,
             comm_r, comm_l, ssem_r, rsem_r, ssem_l, rsem_l,
             credit_r, credit_l):
        my = lax.axis_index("i")
        right = (my + 1) % N_DEV
        left = (my - 1) % N_DEV

        comms = (comm_r, comm_l)
        ssems = (ssem_r, ssem_l)
        rsems = (rsem_r, rsem_l)
        credits = (credit_r, credit_l)
        dsts = (right, left)
        srcs = (left, right)

        barrier_sem = pltpu.get_barrier_semaphore()
        for nbr in (left, right):
            pl.semaphore_signal(
                barrier_sem, inc=1,
                device_id=(nbr,), device_id_type=pl.DeviceIdType.MESH,
            )
        pl.semaphore_wait(barrier_sem, 2)

        def partial(chunk_idx, ring, j):
            rows = x_ref[pl.ds(chunk_idx * m_per, m_per), :]
            col0 = ring * n2 + j * n4
            return lax.dot_general(
                rows, w_ref[:, col0:col0 + n4],
                (((1,), (0,)), ((), ())),
                preferred_element_type=jnp.int32,
            )

        def chunk_at(ring, h):
            if ring == 0:
                return (my - 2 - h) % N_DEV
            return (my + 2 + h) % N_DEV

        def desc(ring, s_slot, r_slot, j):
            return pltpu.make_async_remote_copy(
                src_ref=comms[ring].at[s_slot, j],
                dst_ref=comms[ring].at[r_slot, j],
                send_sem=ssems[ring].at[s_slot, j],
                recv_sem=rsems[ring].at[r_slot, j],
                device_id=(dsts[ring],),
                device_id_type=pl.DeviceIdType.MESH,
            )

        scale = sx_ref[0] * sw_ref[0]

        for j in range(S):
            for ring in (0, 1):
                comms[ring][0, j, :, :] = partial(chunk_at(ring, -1), ring, j)
                desc(ring, 0, 1, j).start()

        for h in range(N_DEV - 1):
            s_slot = h % 2
            r_slot = (h + 1) % 2
            for j in range(S):
                for ring in (0, 1):
                    part = partial(chunk_at(ring, h), ring, j)
                    desc(ring, s_slot, r_slot, j).wait()
                    if h < N_DEV - 2:
                        pl.semaphore_signal(
                            credits[ring], inc=1,
                            device_id=(srcs[ring],),
                            device_id_type=pl.DeviceIdType.MESH,
                        )
                    if h < N_DEV - 2:
                        comms[ring][r_slot, j, :, :] = (
                            comms[ring][r_slot, j, :, :] + part)
                        pl.semaphore_wait(credits[ring], 1)
                        desc(ring, r_slot, s_slot, j).start()
                    else:
                        col0 = ring * n2 + j * n4
                        acc = comms[ring][r_slot, j, :, :] + part
                        out_ref[:, col0:col0 + n4] = jnp.maximum(
                            acc.astype(jnp.float32) * scale, 0.0)

    return pl.pallas_call(
        body,
        out_shape=jax.ShapeDtypeStruct((m_per, n), jnp.float32),
        in_specs=[
            pl.BlockSpec(memory_space=pltpu.VMEM),
            pl.BlockSpec(memory_space=pltpu.VMEM),
            pl.BlockSpec(memory_space=pltpu.SMEM),
            pl.BlockSpec(memory_space=pltpu.SMEM),
        ],
        out_specs=pl.BlockSpec(memory_space=pltpu.VMEM),
        scratch_shapes=[
            pltpu.VMEM((2, S, m_per, n4), jnp.int32),
            pltpu.VMEM((2, S, m_per, n4), jnp.int32),
            pltpu.SemaphoreType.DMA((2, S)),
            pltpu.SemaphoreType.DMA((2, S)),
            pltpu.SemaphoreType.DMA((2, S)),
            pltpu.SemaphoreType.DMA((2, S)),
            pltpu.SemaphoreType.REGULAR,
            pltpu.SemaphoreType.REGULAR,
        ],
        compiler_params=pltpu.CompilerParams(collective_id=0),
    )(x, w_mat, scale_x, scale_w)


# device time: 689946 ns/iter; 2.0837x vs baseline; 1.0004x over previous
import jax
import jax.numpy as jnp
from jax import lax
from jax.experimental import pallas as pl
from jax.experimental.pallas import tpu as pltpu

N_DEV = 16
S = 4


def kernel(x, w_mat, scale_x, scale_w):
    m_total, k = x.shape
    _, n = w_mat.shape
    m_per = m_total // N_DEV
    n2 = n // 2
    n4 = n2 // S

    def body(x_ref, w_ref, sx_ref, sw_ref, out_ref,
             comm_r, comm_l, ssem_r, rsem_r, ssem_l, rsem_l,
             credit_r, credit_l):
        my = lax.axis_index("i")
        right = (my + 1) % N_DEV
        left = (my - 1) % N_DEV

        comms = (comm_r, comm_l)
        ssems = (ssem_r, ssem_l)
        rsems = (rsem_r, rsem_l)
        credits = (credit_r, credit_l)
        dsts = (right, left)
        srcs = (left, right)

        barrier_sem = pltpu.get_barrier_semaphore()
        for nbr in (left, right):
            pl.semaphore_signal(
                barrier_sem, inc=1,
                device_id=(nbr,), device_id_type=pl.DeviceIdType.MESH,
            )
        pl.semaphore_wait(barrier_sem, 2)

        def partial(chunk_idx, ring, j):
            rows = x_ref[pl.ds(chunk_idx * m_per, m_per), :]
            col0 = ring * n2 + j * n4
            return lax.dot_general(
                rows, w_ref[:, col0:col0 + n4],
                (((1,), (0,)), ((), ())),
                preferred_element_type=jnp.int32,
            )

        def chunk_at(ring, h):
            if ring == 0:
                return (my - 2 - h) % N_DEV
            return (my + 2 + h) % N_DEV

        def desc(ring, s_slot, r_slot, j):
            return pltpu.make_async_remote_copy(
                src_ref=comms[ring].at[s_slot, j],
                dst_ref=comms[ring].at[r_slot, j],
                send_sem=ssems[ring].at[s_slot, j],
                recv_sem=rsems[ring].at[r_slot, j],
                device_id=(dsts[ring],),
                device_id_type=pl.DeviceIdType.MESH,
            )

        scale = sx_ref[0] * sw_ref[0]

        for j in range(S):
            for ring in (0, 1):
                comms[ring][0, j, :, :] = partial(chunk_at(ring, -1), ring, j)
                desc(ring, 0, 1, j).start()

        for h in range(N_DEV - 1):
            s_slot = h % 2
            r_slot = (h + 1) % 2
            for j in range(S):
                for ring in (0, 1):
                    part = partial(chunk_at(ring, h), ring, j)
                    desc(ring, s_slot, r_slot, j).wait()
                    if h < N_DEV - 2:
                        pl.semaphore_signal(
                            credits[ring], inc=1,
                            device_id=(srcs[ring],),
                            device_id_type=pl.DeviceIdType.MESH,
                        )
                    if h < N_DEV - 2:
                        comms[ring][r_slot, j, :, :] = (
                            comms[ring][r_slot, j, :, :] + part)
                        pl.semaphore_wait(credits[ring], 1)
                        desc(ring, r_slot, s_slot, j).start()
                    else:
                        col0 = ring * n2 + j * n4
                        acc = comms[ring][r_slot, j, :, :] + part
                        out_ref[:, col0:col0 + n4] = jnp.maximum(
                            acc.astype(jnp.float32) * scale, 0.0)

    return pl.pallas_call(
        body,
        out_shape=jax.ShapeDtypeStruct((m_per, n), jnp.float32),
        in_specs=[
            pl.BlockSpec(memory_space=pltpu.VMEM),
            pl.BlockSpec(memory_space=pltpu.VMEM),
            pl.BlockSpec(memory_space=pltpu.SMEM),
            pl.BlockSpec(memory_space=pltpu.SMEM),
        ],
        out_specs=pl.BlockSpec(memory_space=pltpu.VMEM),
        scratch_shapes=[
            pltpu.VMEM((2, S, m_per, n4), jnp.int32),
            pltpu.VMEM((2, S, m_per, n4), jnp.int32),
            pltpu.SemaphoreType.DMA((2, S)),
            pltpu.SemaphoreType.DMA((2, S)),
            pltpu.SemaphoreType.DMA((2, S)),
            pltpu.SemaphoreType.DMA((2, S)),
            pltpu.SemaphoreType.REGULAR,
            pltpu.SemaphoreType.REGULAR,
        ],
        compiler_params=pltpu.CompilerParams(collective_id=0),
    )(x, w_mat, scale_x, scale_w)
